# Initial kernel scaffold; baseline (speedup 1.0000x reference)
import jax
import jax.numpy as jnp
from jax.experimental import pallas as pl


def kernel(x, group_list, w):
    raise NotImplementedError("write your pallas kernel here")


# speedup vs baseline: 4.1544x; 4.1544x over previous
import jax
import jax.numpy as jnp
from jax.experimental import pallas as pl
from jax.experimental.pallas import tpu as pltpu

_E, _H, _D = 8, 768, 768
_T = 2048
_BT = 256
_NB = _T // _BT


def _gmm_body(blk_ref, grp_ref, x_ref, w_ref, o_ref):
    b = pl.program_id(0)
    e_lo = blk_ref[b, 0]
    e_hi = blk_ref[b, 1]
    xb = x_ref[...]
    base = b * _BT
    row = jax.lax.broadcasted_iota(jnp.int32, (_BT, 1), 0) + base

    def body(e, acc):
        s = grp_ref[e, 0]
        t = grp_ref[e, 1]
        mask = (row >= s) & (row < t)
        xm = jnp.where(mask, xb, jnp.zeros_like(xb))
        return acc + jnp.dot(xm, w_ref[e], preferred_element_type=jnp.float32)

    acc = jax.lax.fori_loop(
        e_lo, e_hi + 1, body, jnp.zeros((_BT, _D), jnp.float32)
    )
    o_ref[...] = acc


def kernel(x, group_list, w):
    counts = group_list.astype(jnp.int32)
    ends = jnp.cumsum(counts)
    starts = ends - counts
    grp = jnp.stack([starts, ends], axis=1)

    block_first = jnp.arange(_NB, dtype=jnp.int32) * _BT
    block_last = block_first + (_BT - 1)
    e_lo = jnp.searchsorted(ends, block_first, side="right").astype(jnp.int32)
    e_hi = jnp.searchsorted(ends, block_last, side="right").astype(jnp.int32)
    e_hi = jnp.minimum(e_hi, _E - 1)
    blk = jnp.stack([e_lo, e_hi], axis=1)

    grid_spec = pltpu.PrefetchScalarGridSpec(
        num_scalar_prefetch=2,
        grid=(_NB,),
        in_specs=[
            pl.BlockSpec((_BT, _H), lambda i, blk, grp: (i, 0)),
            pl.BlockSpec((_E, _H, _D), lambda i, blk, grp: (0, 0, 0)),
        ],
        out_specs=pl.BlockSpec((_BT, _D), lambda i, blk, grp: (i, 0)),
    )

    return pl.pallas_call(
        _gmm_body,
        grid_spec=grid_spec,
        out_shape=jax.ShapeDtypeStruct((_T, _D), jnp.float32),
    )(blk, grp, x, w)
